# Initial kernel scaffold; baseline (speedup 1.0000x reference)
import jax
import jax.numpy as jnp
from jax.experimental import pallas as pl


def kernel(x, edge_index, W1, b1, W2, b2):
    raise NotImplementedError("write your pallas kernel here")


# speedup vs baseline: 15.7234x; 15.7234x over previous
import functools

import jax
import jax.numpy as jnp
from jax import lax
from jax.experimental import pallas as pl
from jax.experimental.pallas import tpu as pltpu
from jax.experimental.pallas import tpu_sc as plsc

N = 10000
E = 160000
IN_C = 256
HID = 256
NCLS = 8

NC = 2
NS = 16
LANES = 16

CH = 128
EROWS = 1280
EPAD = EROWS * CH
NACC = 10240
NDUMMY = NACC - N
ACC_T = NACC // NS

_HIGHEST = lax.Precision.HIGHEST


def _fill(ref, nrows, ncols, value):
    val = jnp.full((LANES,), value, dtype=ref.dtype)

    def body(i, _):
        for k in range(ncols // LANES):
            ref[i, pl.ds(k * LANES, LANES)] = val
        return 0

    lax.fori_loop(0, nrows, body, 0)


def _zero_acc_chunk(zbuf, acc, tile):
    base = tile * ACC_T
    off = 0
    left = ACC_T
    while left > 0:
        sz = min(CH, left)
        pltpu.sync_copy(zbuf.at[pl.ds(0, sz)], acc.at[pl.ds(base + off, sz)])
        off += sz
        left -= sz


def _make_deg_kernel():
    mesh = plsc.VectorSubcoreMesh(core_axis_name="c", subcore_axis_name="s")
    rows_per = EROWS // (NC * NS)

    @functools.partial(
        pl.kernel,
        out_type=jax.ShapeDtypeStruct((NC, NACC, LANES), jnp.float32),
        mesh=mesh,
        scratch_types=[
            pltpu.VMEM_SHARED((NACC, LANES), jnp.float32),
            pltpu.VMEM((rows_per, CH), jnp.int32),
            pltpu.VMEM((CH, LANES), jnp.float32),
            pltpu.VMEM((CH, LANES), jnp.float32),
        ],
        compiler_params=pltpu.CompilerParams(use_tc_tiling_on_sc=False),
    )
    def deg_kernel(dst_hbm, p_out, acc, dstb, ones, zbuf):
        c = lax.axis_index("c")
        s = lax.axis_index("s")
        wid = s * NC + c

        _fill(ones, CH, LANES, 1.0)
        _fill(zbuf, CH, LANES, 0.0)
        _zero_acc_chunk(zbuf, acc, s)
        plsc.subcore_barrier()

        pltpu.sync_copy(dst_hbm.at[pl.ds(wid * rows_per, rows_per)], dstb)

        def body(j, _):
            pltpu.sync_copy(ones, acc.at[dstb.at[j]], add=True)
            return 0

        lax.fori_loop(0, rows_per, body, 0)
        plsc.subcore_barrier()

        pltpu.sync_copy(acc.at[pl.ds(s * ACC_T, ACC_T)],
                        p_out.at[c, pl.ds(s * ACC_T, ACC_T)])

    return deg_kernel


def _make_mp_kernel(feat):
    mesh = plsc.VectorSubcoreMesh(core_axis_name="c", subcore_axis_name="s")
    rows_per = EROWS // NS

    @functools.partial(
        pl.kernel,
        out_type=jax.ShapeDtypeStruct((NC, NACC, feat), jnp.float32),
        mesh=mesh,
        scratch_types=[
            pltpu.VMEM_SHARED((NACC, feat), jnp.float32),
            pltpu.VMEM((rows_per, CH), jnp.int32),
            pltpu.VMEM((rows_per, CH), jnp.int32),
            pltpu.VMEM((CH, feat), jnp.float32),
            pltpu.SemaphoreType.DMA,
        ],
    )
    def mp_kernel(hp, src_hbm, dst_hbm, out, acc, srcb, dstb, rows, sem):
        c = lax.axis_index("c")
        s = lax.axis_index("s")

        _fill(rows, CH, feat, 0.0)
        _zero_acc_chunk(rows, acc, s)
        plsc.subcore_barrier()

        pltpu.sync_copy(src_hbm.at[pl.ds(s * rows_per, rows_per)], srcb)
        pltpu.sync_copy(dst_hbm.at[pl.ds(s * rows_per, rows_per)], dstb)
        hp_c = hp.at[c]

        def body(j, _):
            pltpu.async_copy(hp_c.at[srcb.at[j]], rows, sem).wait()
            pltpu.sync_copy(rows, acc.at[dstb.at[j]], add=True)
            return 0

        lax.fori_loop(0, rows_per, body, 0)
        plsc.subcore_barrier()

        pltpu.sync_copy(acc.at[pl.ds(s * ACC_T, ACC_T)],
                        out.at[c, pl.ds(s * ACC_T, ACC_T)])

    return mp_kernel


def _make_mp16_kernel():
    feat = LANES
    mesh = plsc.VectorSubcoreMesh(core_axis_name="c", subcore_axis_name="s")
    rows_per = EROWS // (NC * NS)

    @functools.partial(
        pl.kernel,
        out_type=jax.ShapeDtypeStruct((NC, NACC, feat), jnp.float32),
        mesh=mesh,
        scratch_types=[
            pltpu.VMEM_SHARED((NACC, feat), jnp.float32),
            pltpu.VMEM((rows_per, CH), jnp.int32),
            pltpu.VMEM((rows_per, CH), jnp.int32),
            pltpu.VMEM((CH, feat), jnp.float32),
            pltpu.SemaphoreType.DMA,
        ],
        compiler_params=pltpu.CompilerParams(use_tc_tiling_on_sc=False),
    )
    def mp16_kernel(gp, src_hbm, dst_hbm, out, acc, srcb, dstb, rows, sem):
        c = lax.axis_index("c")
        s = lax.axis_index("s")
        wid = s * NC + c

        _fill(rows, CH, feat, 0.0)
        _zero_acc_chunk(rows, acc, s)
        plsc.subcore_barrier()

        pltpu.sync_copy(src_hbm.at[pl.ds(wid * rows_per, rows_per)], srcb)
        pltpu.sync_copy(dst_hbm.at[pl.ds(wid * rows_per, rows_per)], dstb)

        def body(j, _):
            pltpu.async_copy(gp.at[srcb.at[j]], rows, sem).wait()
            pltpu.sync_copy(rows, acc.at[dstb.at[j]], add=True)
            return 0

        lax.fori_loop(0, rows_per, body, 0)
        plsc.subcore_barrier()

        pltpu.sync_copy(acc.at[pl.ds(s * ACC_T, ACC_T)],
                        out.at[c, pl.ds(s * ACC_T, ACC_T)])

    return mp16_kernel


_RB = 1000


def _lin1_body(x_ref, w_ref, p0_ref, p1_ref, hp_ref, d_ref):
    deg = p0_ref[0][:, 0:1] + p1_ref[0][:, 0:1] + 1.0
    d = lax.rsqrt(deg)
    h = jnp.dot(x_ref[...], w_ref[...], precision=_HIGHEST,
                preferred_element_type=jnp.float32)
    hp_ref[0] = h * d
    d_ref[...] = jnp.broadcast_to(d, (_RB, 8))


def _lin2_body(a0_ref, a1_ref, hp0_ref, hp1_ref, d_ref, b1_ref, w2_ref, gp_ref):
    d = d_ref[:, 0:1]
    h = jnp.concatenate(
        [a0_ref[0] + hp0_ref[0], a1_ref[0] + hp1_ref[0]], axis=1)
    h = jnp.maximum(h * d + b1_ref[...], 0.0)
    g = jnp.dot(h, w2_ref[...], precision=_HIGHEST,
                preferred_element_type=jnp.float32)
    gp_ref[...] = jnp.concatenate(
        [g * d, jnp.zeros((_RB, LANES - NCLS), jnp.float32)], axis=1)


def _final_body(q0_ref, q1_ref, gp_ref, d_ref, b2_ref, out_ref):
    d = d_ref[:, 0:1]
    tot = q0_ref[0] + q1_ref[0] + gp_ref[...]
    out_ref[...] = d * tot[:, :NCLS] + b2_ref[...]


def _row_spec(cols):
    return pl.BlockSpec((_RB, cols), lambda i: (i, 0))


def _half_spec(half, cols):
    return pl.BlockSpec((1, _RB, cols), lambda i: (half, i, 0))


def _full_spec(rows, cols):
    return pl.BlockSpec((rows, cols), lambda i: (0, 0))


def kernel(x, edge_index, W1, b1, W2, b2):
    src = edge_index[0]
    dst = edge_index[1]
    npad = EPAD - E
    ar = jnp.arange(npad, dtype=jnp.int32)
    pad_src = (ar * 2003) % N
    pad_dst = N + (ar % NDUMMY)
    srcp = jnp.concatenate([src, pad_src]).reshape(EROWS, CH)
    dstp = jnp.concatenate([dst, pad_dst]).reshape(EROWS, CH)

    p = _make_deg_kernel()(dstp)

    hp, d8 = pl.pallas_call(
        _lin1_body,
        grid=(2, N // _RB),
        in_specs=[
            pl.BlockSpec((_RB, IN_C), lambda h, i: (i, 0)),
            pl.BlockSpec((IN_C, HID // 2), lambda h, i: (0, h)),
            pl.BlockSpec((1, _RB, LANES), lambda h, i: (0, i, 0)),
            pl.BlockSpec((1, _RB, LANES), lambda h, i: (1, i, 0)),
        ],
        out_specs=[
            pl.BlockSpec((1, _RB, HID // 2), lambda h, i: (h, i, 0)),
            pl.BlockSpec((_RB, 8), lambda h, i: (i, 0)),
        ],
        out_shape=[
            jax.ShapeDtypeStruct((NC, N, HID // 2), jnp.float32),
            jax.ShapeDtypeStruct((N, 8), jnp.float32),
        ],
    )(x, W1, p, p)

    a = _make_mp_kernel(HID // 2)(hp, srcp, dstp)

    gp = pl.pallas_call(
        _lin2_body,
        grid=(N // _RB,),
        in_specs=[
            _half_spec(0, HID // 2),
            _half_spec(1, HID // 2),
            _half_spec(0, HID // 2),
            _half_spec(1, HID // 2),
            _row_spec(8),
            _full_spec(1, HID),
            _full_spec(HID, NCLS),
        ],
        out_specs=_row_spec(LANES),
        out_shape=jax.ShapeDtypeStruct((N, LANES), jnp.float32),
    )(a, a, hp, hp, d8, b1.reshape(1, HID), W2)

    q = _make_mp16_kernel()(gp, srcp, dstp)

    out = pl.pallas_call(
        _final_body,
        grid=(N // _RB,),
        in_specs=[
            _half_spec(0, LANES),
            _half_spec(1, LANES),
            _row_spec(LANES),
            _row_spec(8),
            _full_spec(1, NCLS),
        ],
        out_specs=_row_spec(NCLS),
        out_shape=jax.ShapeDtypeStruct((N, NCLS), jnp.float32),
    )(q, q, gp, d8, b2.reshape(1, NCLS))

    return out
